# baseline (speedup 1.0000x reference)
import functools

import jax
import jax.numpy as jnp
from jax import lax
from jax.experimental import pallas as pl
from jax.experimental.pallas import tpu as pltpu
from jax.experimental.pallas import tpu_sc as plsc

B = 16384
E = 32
F = 32
H = 256
NROWS = 1_000_000

_info = plsc.get_sparse_core_info()
_NC, _NS = _info.num_cores, _info.num_subcores
_NW = _NC * _NS
_BPW = B // _NW

_sc_mesh = plsc.VectorSubcoreMesh(core_axis_name="c", subcore_axis_name="s")


@functools.partial(
    pl.kernel,
    out_type=(
        jax.ShapeDtypeStruct((E * B,), jnp.float32),
        jax.ShapeDtypeStruct((E * B,), jnp.float32),
    ),
    mesh=_sc_mesh,
    scratch_types=[
        pltpu.VMEM((_BPW,), jnp.int32),
        pltpu.VMEM((_BPW,), jnp.float32),
        pltpu.VMEM((_BPW,), jnp.int32),
        pltpu.VMEM((_BPW,), jnp.float32),
        pltpu.SemaphoreType.DMA,
        pltpu.SemaphoreType.DMA,
    ],
)
def _gather_sc(uidx_hbm, uflat_hbm, ridx_hbm, rflat_hbm, uout_hbm, rout_hbm,
               uidx_v, urows_v, ridx_v, rrows_v, usem, rsem):
    wid = lax.axis_index("s") * _NC + lax.axis_index("c")
    base = wid * _BPW
    for e in range(E):
        off = e * B + base
        pltpu.sync_copy(uidx_hbm.at[pl.ds(off, _BPW)], uidx_v)
        pltpu.sync_copy(ridx_hbm.at[pl.ds(off, _BPW)], ridx_v)
        ucp = pltpu.async_copy(uflat_hbm.at[uidx_v], urows_v, usem)
        rcp = pltpu.async_copy(rflat_hbm.at[ridx_v], rrows_v, rsem)
        ucp.wait()
        rcp.wait()
        pltpu.sync_copy(urows_v, uout_hbm.at[pl.ds(off, _BPW)])
        pltpu.sync_copy(rrows_v, rout_hbm.at[pl.ds(off, _BPW)])


_BN = 2048


def _dotT(xT, w):
    return lax.dot_general(xT, w, (((0,), (0,)), ((), ())),
                           preferred_element_type=jnp.float32)


def _towers_body(uembT, ufeatT, rembT, rfeatT,
                 uA1, uB1, ub1, uW2t, ub2, uW3t, ub3,
                 rA1, rB1, rb1, rW2t, rb2, rW3t, rb3, out):
    def tower(embT, featT, A1, B1, b1, W2t, b2, W3t, b3):
        h = _dotT(embT[...], A1[...]) + _dotT(featT[...], B1[...]) + b1[...]
        h = jnp.maximum(h, 0.0)
        h = jnp.dot(h, W2t[...], preferred_element_type=jnp.float32) + b2[...]
        h = jnp.maximum(h, 0.0)
        o = jnp.dot(h, W3t[...], preferred_element_type=jnp.float32) + b3[...]
        n = jnp.sqrt(jnp.sum(o * o, axis=1, keepdims=True))
        return o / jnp.maximum(n, 1e-12)

    u = tower(uembT, ufeatT, uA1, uB1, ub1, uW2t, ub2, uW3t, ub3)
    r = tower(rembT, rfeatT, rA1, rB1, rb1, rW2t, rb2, rW3t, rb3)
    out[...] = jnp.sum(u * r, axis=1, keepdims=True)


def _full(shape):
    return pl.BlockSpec(shape, lambda i: (0,) * len(shape))


_towers_tc = pl.pallas_call(
    _towers_body,
    grid=(B // _BN,),
    in_specs=[
        pl.BlockSpec((E, _BN), lambda i: (0, i)),
        pl.BlockSpec((F, _BN), lambda i: (0, i)),
        pl.BlockSpec((E, _BN), lambda i: (0, i)),
        pl.BlockSpec((F, _BN), lambda i: (0, i)),
        _full((E, H)), _full((F, H)), _full((1, H)),
        _full((H, H)), _full((1, H)),
        _full((H, E)), _full((1, E)),
        _full((E, H)), _full((F, H)), _full((1, H)),
        _full((H, H)), _full((1, H)),
        _full((H, E)), _full((1, E)),
    ],
    out_specs=pl.BlockSpec((_BN, 1), lambda i: (i, 0)),
    out_shape=jax.ShapeDtypeStruct((B, 1), jnp.float32),
)


def kernel(user_ids, user_features, recipe_ids, recipe_features,
           user_table, recipe_table,
           uW1, ub1, uW2, ub2, uW3, ub3,
           rW1, rb1, rW2, rb2, rW3, rb3):
    uids = user_ids.astype(jnp.int32)
    rids = recipe_ids.astype(jnp.int32)
    offs = (jnp.arange(E, dtype=jnp.int32) * NROWS).reshape(E, 1)
    uidxflat = (uids.reshape(1, B) + offs).reshape(E * B)
    ridxflat = (rids.reshape(1, B) + offs).reshape(E * B)
    uflat = user_table.T.reshape(E * NROWS)
    rflat = recipe_table.T.reshape(E * NROWS)
    uout, rout = _gather_sc(uidxflat, uflat, ridxflat, rflat)
    uembT = uout.reshape(E, B)
    rembT = rout.reshape(E, B)
    scores = _towers_tc(
        uembT, user_features.T, rembT, recipe_features.T,
        uW1[:, :E].T, uW1[:, E:].T, ub1.reshape(1, H),
        uW2.T, ub2.reshape(1, H), uW3.T, ub3.reshape(1, E),
        rW1[:, :E].T, rW1[:, E:].T, rb1.reshape(1, H),
        rW2.T, rb2.reshape(1, H), rW3.T, rb3.reshape(1, E),
    )
    return scores.reshape(B)


# speedup vs baseline: 18.6026x; 18.6026x over previous
import functools

import jax
import jax.numpy as jnp
from jax import lax
from jax.experimental import pallas as pl
from jax.experimental.pallas import tpu as pltpu
from jax.experimental.pallas import tpu_sc as plsc

B = 16384
E = 32
F = 32
H = 256
NROWS = 1_000_000

_info = plsc.get_sparse_core_info()
_NC, _NS = _info.num_cores, _info.num_subcores
_NW = _NC * _NS
_BPW = B // _NW

_W = 131072
_NCOLS = 7
_WREM = 82432
_TAIL0 = _NCOLS * _W + _WREM
_S = 8 * _W


def _repack_body(utabT, utail, uflat,
                 big0, big1, rem0, rem1, tl0, tl1, insem0, insem1, outsem):
    bufs = {_W: (big0, big1), _WREM: (rem0, rem1), 128: (tl0, tl1)}
    insems = {id(big0): insem0, id(rem0): insem0, id(tl0): insem0,
              id(big1): insem1, id(rem1): insem1, id(tl1): insem1}
    chunks = ([(c * _W, _W) for c in range(_NCOLS)]
              + [(_NCOLS * _W, _WREM), (_TAIL0, 128)])
    work = []
    for tab, tail, flat2 in ((utabT, utail, uflat),):
        for g in range(E // 8):
            for off, width in chunks:
                work.append((tab, tail, flat2, g, off, width))

    nslot = {}

    def pick(width):
        k = nslot.get(width, 0)
        nslot[width] = k + 1
        return bufs[width][k % 2]

    def start(item):
        tab, tail, _, g, off, width = item
        buf = pick(width)
        if off == _TAIL0:
            src = tail.at[pl.ds(g * 8, 8), :]
        else:
            src = tab.at[pl.ds(g * 8, 8), pl.ds(off, width)]
        pltpu.async_copy(src, buf, insems[id(buf)])
        return buf

    def write_out(buf, item):
        _, _, flat2, g, off, width = item
        pltpu.make_async_copy(buf, buf, insems[id(buf)]).wait()
        for s in range(8):
            dst = flat2.at[pl.ds((g * 8 + s) * _S + off, width)]
            pltpu.async_copy(buf.at[s], dst, outsem)

    def drain_out(item):
        _, _, flat2, g, off, width = item
        for s in range(8):
            dst = flat2.at[pl.ds((g * 8 + s) * _S + off, width)]
            pltpu.make_async_copy(dst, dst, outsem).wait()

    cur = start(work[0])
    for k, item in enumerate(work):
        nxt = start(work[k + 1]) if k + 1 < len(work) else None
        write_out(cur, item)
        drain_out(item)
        cur = nxt


_repack_tc = pl.pallas_call(
    _repack_body,
    in_specs=[
        pl.BlockSpec(memory_space=pl.ANY),
        pl.BlockSpec(memory_space=pl.ANY),
    ],
    out_specs=[
        pl.BlockSpec(memory_space=pl.ANY),
    ],
    out_shape=[
        jax.ShapeDtypeStruct((E * _S,), jnp.float32),
    ],
    scratch_shapes=[
        pltpu.VMEM((8, _W), jnp.float32),
        pltpu.VMEM((8, _W), jnp.float32),
        pltpu.VMEM((8, _WREM), jnp.float32),
        pltpu.VMEM((8, _WREM), jnp.float32),
        pltpu.VMEM((8, 128), jnp.float32),
        pltpu.VMEM((8, 128), jnp.float32),
        pltpu.SemaphoreType.DMA,
        pltpu.SemaphoreType.DMA,
        pltpu.SemaphoreType.DMA,
    ],
)


_sc_mesh = plsc.VectorSubcoreMesh(core_axis_name="c", subcore_axis_name="s")


@functools.partial(
    pl.kernel,
    out_type=jax.ShapeDtypeStruct((E * B,), jnp.float32),
    mesh=_sc_mesh,
    scratch_types=[
        [pltpu.VMEM((_BPW,), jnp.int32) for _ in range(2)],
        [pltpu.VMEM((_BPW,), jnp.float32) for _ in range(2)],
        [pltpu.SemaphoreType.DMA for _ in range(2)],
    ],
)
def _gather_sc(uids_hbm, uflat_hbm, uout_hbm, uidx_v, urows_v, usem):
    wid = lax.axis_index("s") * _NC + lax.axis_index("c")
    base = wid * _BPW
    pltpu.sync_copy(uids_hbm.at[pl.ds(base, _BPW)], uidx_v[0])
    stride = jnp.full((16,), _S, jnp.int32)
    for k in range(_BPW // 16):
        sl = pl.ds(k * 16, 16)
        uidx_v[1][sl] = uidx_v[0][sl] + stride
    pltpu.async_copy(uflat_hbm.at[uidx_v[0]], urows_v[0], usem[0])
    for e in range(E):
        cur = e % 2
        if e + 1 < E:
            if e >= 1:
                for k in range(_BPW // 16):
                    sl = pl.ds(k * 16, 16)
                    uidx_v[1 - cur][sl] = uidx_v[1 - cur][sl] + stride + stride
            pltpu.async_copy(uflat_hbm.at[uidx_v[1 - cur]],
                             urows_v[1 - cur], usem[1 - cur])
        pltpu.make_async_copy(uout_hbm.at[pl.ds(0, _BPW)],
                              urows_v[cur], usem[cur]).wait()
        pltpu.sync_copy(urows_v[cur], uout_hbm.at[pl.ds(e * B + base, _BPW)])


_BN = 2048


def _dotT(xT, w):
    return lax.dot_general(xT, w, (((0,), (0,)), ((), ())),
                           preferred_element_type=jnp.float32)


def _towers_body(uembT, ufeatT, rembT, rfeatT,
                 uA1, uB1, ub1, uW2t, ub2, uW3t, ub3,
                 rA1, rB1, rb1, rW2t, rb2, rW3t, rb3, out):
    def tower(embT, featT, A1, B1, b1, W2t, b2, W3t, b3):
        h = _dotT(embT[...], A1[...]) + _dotT(featT[...], B1[...]) + b1[...]
        h = jnp.maximum(h, 0.0)
        h = jnp.dot(h, W2t[...], preferred_element_type=jnp.float32) + b2[...]
        h = jnp.maximum(h, 0.0)
        o = jnp.dot(h, W3t[...], preferred_element_type=jnp.float32) + b3[...]
        n = jnp.sqrt(jnp.sum(o * o, axis=1, keepdims=True))
        return o / jnp.maximum(n, 1e-12)

    u = tower(uembT, ufeatT, uA1, uB1, ub1, uW2t, ub2, uW3t, ub3)
    r = tower(rembT, rfeatT, rA1, rB1, rb1, rW2t, rb2, rW3t, rb3)
    out[...] = jnp.sum(u * r, axis=1, keepdims=True)


def _full(shape):
    return pl.BlockSpec(shape, lambda i: (0,) * len(shape))


_towers_tc = pl.pallas_call(
    _towers_body,
    grid=(B // _BN,),
    in_specs=[
        pl.BlockSpec((E, _BN), lambda i: (0, i)),
        pl.BlockSpec((F, _BN), lambda i: (0, i)),
        pl.BlockSpec((E, _BN), lambda i: (0, i)),
        pl.BlockSpec((F, _BN), lambda i: (0, i)),
        _full((E, H)), _full((F, H)), _full((1, H)),
        _full((H, H)), _full((1, H)),
        _full((H, E)), _full((1, E)),
        _full((E, H)), _full((F, H)), _full((1, H)),
        _full((H, H)), _full((1, H)),
        _full((H, E)), _full((1, E)),
    ],
    out_specs=pl.BlockSpec((_BN, 1), lambda i: (i, 0)),
    out_shape=jax.ShapeDtypeStruct((B, 1), jnp.float32),
)


def kernel(user_ids, user_features, recipe_ids, recipe_features,
           user_table, recipe_table,
           uW1, ub1, uW2, ub2, uW3, ub3,
           rW1, rb1, rW2, rb2, rW3, rb3):
    uids = user_ids.astype(jnp.int32)
    rids = recipe_ids.astype(jnp.int32)
    utail = jnp.pad(user_table.T[:, _TAIL0:], ((0, 0), (0, 128 - (NROWS - _TAIL0))))
    rtail = jnp.pad(recipe_table.T[:, _TAIL0:], ((0, 0), (0, 128 - (NROWS - _TAIL0))))
    [uflat] = _repack_tc(user_table.T, utail)
    uout = _gather_sc(uids, uflat)
    [rflat] = _repack_tc(recipe_table.T, rtail)
    rout = _gather_sc(rids, rflat)
    uembT = uout.reshape(E, B)
    rembT = rout.reshape(E, B)
    scores = _towers_tc(
        uembT, user_features.T, rembT, recipe_features.T,
        uW1[:, :E].T, uW1[:, E:].T, ub1.reshape(1, H),
        uW2.T, ub2.reshape(1, H), uW3.T, ub3.reshape(1, E),
        rW1[:, :E].T, rW1[:, E:].T, rb1.reshape(1, H),
        rW2.T, rb2.reshape(1, H), rW3.T, rb3.reshape(1, E),
    )
    return scores.reshape(B)
